# baseline (speedup 1.0000x reference)
import functools

import jax
import jax.numpy as jnp
from jax import lax
from jax.experimental import pallas as pl
from jax.experimental.pallas import tpu as pltpu
from jax.experimental.pallas import tpu_sc as plsc

N_USER = 5000
N_NODE = 10000
NPAD = 10240
D = 128
NC, NS, L = 2, 16, 16
NTILE = NC * NS
ECHUNK = 128
NCHUNK = 84
EPT = NCHUNK * ECHUNK
EPAD = NTILE * EPT
ROWS_PT = NPAD // NS


def _tc_body(x_ref, w_ref, as_ref, ad_ref, xp_ref, s_ref, d_ref):
    xp = jnp.dot(x_ref[...], w_ref[...], preferred_element_type=jnp.float32)
    xp_ref[...] = xp
    s_ref[...] = jnp.sum(xp * as_ref[...], axis=1, keepdims=True)
    d_ref[...] = jnp.sum(xp * ad_ref[...], axis=1, keepdims=True)


def _tc_layer0(x, W, a_s, a_d):
    R = 1024
    grid = NPAD // R
    return pl.pallas_call(
        _tc_body,
        grid=(grid,),
        in_specs=[
            pl.BlockSpec((R, D), lambda i: (i, 0)),
            pl.BlockSpec((D, D), lambda i: (0, 0)),
            pl.BlockSpec((1, D), lambda i: (0, 0)),
            pl.BlockSpec((1, D), lambda i: (0, 0)),
        ],
        out_specs=[
            pl.BlockSpec((R, D), lambda i: (i, 0)),
            pl.BlockSpec((R, 1), lambda i: (i, 0)),
            pl.BlockSpec((R, 1), lambda i: (i, 0)),
        ],
        out_shape=[
            jax.ShapeDtypeStruct((NPAD, D), jnp.float32),
            jax.ShapeDtypeStruct((NPAD, 1), jnp.float32),
            jax.ShapeDtypeStruct((NPAD, 1), jnp.float32),
        ],
    )(x, W, a_s.reshape(1, D), a_d.reshape(1, D))


def _tc_body_p(p_ref, b_ref, w_ref, as_ref, ad_ref, xp_ref, s_ref, d_ref):
    x = p_ref[0] + p_ref[1] + b_ref[...]
    xp = jnp.dot(x, w_ref[...], preferred_element_type=jnp.float32)
    xp_ref[...] = xp
    s_ref[...] = jnp.sum(xp * as_ref[...], axis=1, keepdims=True)
    d_ref[...] = jnp.sum(xp * ad_ref[...], axis=1, keepdims=True)


def _tc_layer(p, b, W, a_s, a_d):
    R = 1024
    grid = NPAD // R
    return pl.pallas_call(
        _tc_body_p,
        grid=(grid,),
        in_specs=[
            pl.BlockSpec((2, R, D), lambda i: (0, i, 0)),
            pl.BlockSpec((1, D), lambda i: (0, 0)),
            pl.BlockSpec((D, D), lambda i: (0, 0)),
            pl.BlockSpec((1, D), lambda i: (0, 0)),
            pl.BlockSpec((1, D), lambda i: (0, 0)),
        ],
        out_specs=[
            pl.BlockSpec((R, D), lambda i: (i, 0)),
            pl.BlockSpec((R, 1), lambda i: (i, 0)),
            pl.BlockSpec((R, 1), lambda i: (i, 0)),
        ],
        out_shape=[
            jax.ShapeDtypeStruct((NPAD, D), jnp.float32),
            jax.ShapeDtypeStruct((NPAD, 1), jnp.float32),
            jax.ShapeDtypeStruct((NPAD, 1), jnp.float32),
        ],
    )(p, b.reshape(1, D), W, a_s.reshape(1, D), a_d.reshape(1, D))


_MESH = plsc.VectorSubcoreMesh(
    core_axis_name="c", subcore_axis_name="s", num_cores=NC, num_subcores=NS)

_PIPELINE = True


def _sc_layer_body(xp_hbm, as_hbm, ad_hbm, src_hbm, dst_hbm, out_hbm,
                   src_c, dst_c, asg_v, adg_v, deng_v, exw_v, rows_v, zeros_v,
                   as_sh, ad_sh, den_sh, out_sh):
    c = lax.axis_index("c")
    s = lax.axis_index("s")
    z16 = jnp.zeros((L,), jnp.float32)

    for j in range(8):
        zeros_v[pl.ds(16 * j, 16)] = z16

    def zero_rows(r, _):
        for j in range(8):
            rows_v[0, r, pl.ds(16 * j, 16)] = z16
        return 0
    lax.fori_loop(0, ECHUNK, zero_rows, 0)

    row0 = s * ROWS_PT
    pltpu.sync_copy(as_hbm.at[pl.ds(row0, ROWS_PT)],
                    as_sh.at[pl.ds(row0, ROWS_PT)])
    pltpu.sync_copy(ad_hbm.at[pl.ds(row0, ROWS_PT)],
                    ad_sh.at[pl.ds(row0, ROWS_PT)])
    for m in range(ROWS_PT // ECHUNK):
        pltpu.sync_copy(zeros_v, den_sh.at[pl.ds(row0 + m * ECHUNK, ECHUNK)])
        pltpu.sync_copy(rows_v.at[0],
                        out_sh.at[pl.ds(row0 + m * ECHUNK, ECHUNK)])
    plsc.subcore_barrier()

    def run_phase(nch, base0, issue_gathers, wait_gathers, compute,
                  scatter_src, scatter_dst):
        def scoped(**sems):
            isem = [[sems[f"i{q}{d}"] for d in range(2)] for q in range(4)]
            gsem = [[sems[f"g{k}{d}"] for d in range(4)] for k in range(2)]
            ssem = [sems["s0"], sems["s1"]]
            _run_phase(nch, base0, issue_gathers, wait_gathers, compute,
                       scatter_src, scatter_dst, isem, gsem, ssem)
        names = ([f"i{q}{d}" for q in range(4) for d in range(2)]
                 + [f"g{k}{d}" for k in range(2) for d in range(4)]
                 + ["s0", "s1"])
        pl.run_scoped(
            scoped, **{n: pltpu.SemaphoreType.DMA(()) for n in names})

    def _run_phase(nch, base0, issue_gathers, wait_gathers, compute,
                   scatter_src, scatter_dst, isem, gsem, ssem):
        def issue_idx(g, q):
            base = base0 + g * ECHUNK
            pltpu.async_copy(src_hbm.at[pl.ds(base, ECHUNK)], src_c.at[q],
                             isem[q][0])
            pltpu.async_copy(dst_hbm.at[pl.ds(base, ECHUNK)], dst_c.at[q],
                             isem[q][1])

        def wait_idx(g, q):
            base = base0 + g * ECHUNK
            pltpu.make_async_copy(src_hbm.at[pl.ds(base, ECHUNK)],
                                  src_c.at[q], isem[q][0]).wait()
            pltpu.make_async_copy(dst_hbm.at[pl.ds(base, ECHUNK)],
                                  dst_c.at[q], isem[q][1]).wait()

        def issue_scatter(q, k):
            pltpu.async_copy(scatter_src(k), scatter_dst(q), ssem[k],
                             add=True)

        def wait_scatter(q, k):
            pltpu.make_async_copy(scatter_src(k), scatter_dst(q),
                                  ssem[k]).wait()

        if not _PIPELINE:
            def body_sync(g, _):
                base = base0 + g * ECHUNK
                pltpu.sync_copy(src_hbm.at[pl.ds(base, ECHUNK)], src_c.at[0])
                pltpu.sync_copy(dst_hbm.at[pl.ds(base, ECHUNK)], dst_c.at[0])
                issue_gathers(0, 0, gsem[0])
                wait_gathers(0, 0, gsem[0])
                compute(0)
                pltpu.sync_copy(scatter_src(0), scatter_dst(0), add=True)
                return 0
            lax.fori_loop(0, nch, body_sync, 0)
            return

        issue_idx(jnp.int32(0), 0)
        issue_idx(jnp.int32(1), 1)
        wait_idx(jnp.int32(0), 0)
        issue_gathers(0, 0, gsem[0])

        def body4(i, _):
            for k4 in range(4):
                g = 4 * i + k4
                k = k4 % 2
                q = k4
                qn1 = (k4 + 1) % 4
                qn2 = (k4 + 2) % 4
                wait_gathers(q, k, gsem[k])
                compute(k)
                issue_scatter(q, k)

                @pl.when(g + 2 < nch)
                def _():
                    issue_idx(g + 2, qn2)

                @pl.when(g + 1 < nch)
                def _():
                    wait_idx(g + 1, qn1)

                @pl.when(g >= 1)
                def _():
                    wait_scatter((k4 + 3) % 4, 1 - k)

                @pl.when(g + 1 < nch)
                def _():
                    issue_gathers(qn1, 1 - k, gsem[1 - k])
            return 0
        lax.fori_loop(0, nch // 4, body4, 0)
        wait_scatter(3, 1)

    def issue_ga(q, k, sem):
        pltpu.async_copy(as_sh.at[src_c.at[q]], asg_v.at[k], sem[0])
        pltpu.async_copy(ad_sh.at[dst_c.at[q]], adg_v.at[k], sem[1])

    def wait_ga(q, k, sem):
        pltpu.make_async_copy(as_sh.at[src_c.at[q]], asg_v.at[k],
                              sem[0]).wait()
        pltpu.make_async_copy(ad_sh.at[dst_c.at[q]], adg_v.at[k],
                              sem[1]).wait()

    def compute_a(k):
        for j in range(8):
            sl = pl.ds(16 * j, 16)
            al = asg_v[k, sl] + adg_v[k, sl]
            al = jnp.maximum(al, al * 0.2)
            exw_v[k, sl] = jnp.exp(al)

    run_phase(2 * NCHUNK, 2 * s * EPT, issue_ga, wait_ga, compute_a,
              lambda k: exw_v.at[k], lambda q: den_sh.at[dst_c.at[q]])
    plsc.subcore_barrier()

    def issue_gb(q, k, sem):
        pltpu.async_copy(xp_hbm.at[src_c.at[q]], rows_v.at[k], sem[0])
        pltpu.async_copy(as_sh.at[src_c.at[q]], asg_v.at[k], sem[1])
        pltpu.async_copy(ad_sh.at[dst_c.at[q]], adg_v.at[k], sem[2])
        pltpu.async_copy(den_sh.at[dst_c.at[q]], deng_v.at[k], sem[3])

    def wait_gb(q, k, sem):
        pltpu.make_async_copy(xp_hbm.at[src_c.at[q]], rows_v.at[k],
                              sem[0]).wait()
        pltpu.make_async_copy(as_sh.at[src_c.at[q]], asg_v.at[k],
                              sem[1]).wait()
        pltpu.make_async_copy(ad_sh.at[dst_c.at[q]], adg_v.at[k],
                              sem[2]).wait()
        pltpu.make_async_copy(den_sh.at[dst_c.at[q]], deng_v.at[k],
                              sem[3]).wait()

    def compute_b(k):
        for j in range(8):
            sl = pl.ds(16 * j, 16)
            al = asg_v[k, sl] + adg_v[k, sl]
            al = jnp.maximum(al, al * 0.2)
            exw_v[k, sl] = jnp.exp(al) / deng_v[k, sl]

        def scale(e, _):
            ws = plsc.load_gather(exw_v.at[k], [jnp.full((L,), e, jnp.int32)])
            for j in range(8):
                sl = pl.ds(16 * j, 16)
                rows_v[k, e, sl] = rows_v[k, e, sl] * ws
            return 0
        lax.fori_loop(0, ECHUNK, scale, 0)

    run_phase(NCHUNK, (s * NC + c) * EPT, issue_gb, wait_gb, compute_b,
              lambda k: rows_v.at[k], lambda q: out_sh.at[dst_c.at[q]])
    plsc.subcore_barrier()

    pltpu.sync_copy(out_sh.at[pl.ds(row0, ROWS_PT)],
                    out_hbm.at[c].at[pl.ds(row0, ROWS_PT)])


_SC_PARAMS = pltpu.CompilerParams(needs_layout_passes=False)

_sc_layer = pl.kernel(
    _sc_layer_body, mesh=_MESH,
    compiler_params=_SC_PARAMS,
    out_type=jax.ShapeDtypeStruct((NC, NPAD, D), jnp.float32),
    scratch_types=[
        pltpu.VMEM((4, ECHUNK), jnp.int32),
        pltpu.VMEM((4, ECHUNK), jnp.int32),
        pltpu.VMEM((2, ECHUNK), jnp.float32),
        pltpu.VMEM((2, ECHUNK), jnp.float32),
        pltpu.VMEM((2, ECHUNK), jnp.float32),
        pltpu.VMEM((2, ECHUNK), jnp.float32),
        pltpu.VMEM((2, ECHUNK, D), jnp.float32),
        pltpu.VMEM((ECHUNK,), jnp.float32),
        pltpu.VMEM_SHARED((NPAD,), jnp.float32),
        pltpu.VMEM_SHARED((NPAD,), jnp.float32),
        pltpu.VMEM_SHARED((NPAD,), jnp.float32),
        pltpu.VMEM_SHARED((NPAD, D), jnp.float32),
    ],
)


def _final_body(p0_hbm, p1_hbm, b_hbm, uidx_hbm, iidx_hbm,
                uout_hbm, iout_hbm, idx_v, rows0_v, rows1_v, bias_v):
    c = lax.axis_index("c")
    s = lax.axis_index("s")
    wid = s * NC + c
    pltpu.sync_copy(b_hbm, bias_v)
    b16 = [bias_v[pl.ds(16 * j, 16)] for j in range(8)]

    def one(idx_hbm, out_hbm, offset):
        pltpu.sync_copy(idx_hbm.at[wid], idx_v)
        if offset:
            for g in range(8):
                idx_v[pl.ds(16 * g, 16)] = idx_v[pl.ds(16 * g, 16)] + offset
        pltpu.sync_copy(p0_hbm.at[idx_v], rows0_v)
        pltpu.sync_copy(p1_hbm.at[idx_v], rows1_v)

        def addrow(r, _):
            for j in range(8):
                sl = pl.ds(16 * j, 16)
                rows0_v[r, sl] = rows0_v[r, sl] + rows1_v[r, sl] + b16[j]
            return 0
        lax.fori_loop(0, 128, addrow, 0)
        pltpu.sync_copy(rows0_v, out_hbm.at[pl.ds(wid * 128, 128)])

    one(uidx_hbm, uout_hbm, 0)
    one(iidx_hbm, iout_hbm, N_USER)


_final_gather = pl.kernel(
    _final_body, mesh=_MESH,
    compiler_params=_SC_PARAMS,
    out_type=(jax.ShapeDtypeStruct((4096, D), jnp.float32),
              jax.ShapeDtypeStruct((4096, D), jnp.float32)),
    scratch_types=[
        pltpu.VMEM((128,), jnp.int32),
        pltpu.VMEM((128, D), jnp.float32),
        pltpu.VMEM((128, D), jnp.float32),
        pltpu.VMEM((D,), jnp.float32),
    ],
)


def kernel(user_emb, item_emb, Ws, att_src, att_dst, biases, edge_index, user, item):
    x0 = jnp.concatenate([user_emb, item_emb], axis=0)
    x0 = jnp.pad(x0, ((0, NPAD - N_NODE), (0, 0)))

    loops = jnp.arange(N_NODE, dtype=jnp.int32)
    fill = jnp.full((EPAD - N_NODE - edge_index.shape[1],), N_NODE, jnp.int32)
    src_r = jnp.concatenate([edge_index[0].astype(jnp.int32), loops, fill])
    dst_r = jnp.concatenate([edge_index[1].astype(jnp.int32), loops, fill])

    xp, as2, ad2 = _tc_layer0(x0, Ws[0], att_src[0], att_dst[0])
    p = _sc_layer(xp, as2.reshape(NPAD), ad2.reshape(NPAD), src_r, dst_r)
    for l in range(1, Ws.shape[0]):
        xp, as2, ad2 = _tc_layer(p, biases[l - 1], Ws[l], att_src[l], att_dst[l])
        p = _sc_layer(xp, as2.reshape(NPAD), ad2.reshape(NPAD), src_r, dst_r)

    uidx = user.astype(jnp.int32).reshape(NTILE, 128)
    iidx = item.astype(jnp.int32).reshape(NTILE, 128)
    user_out, item_out = _final_gather(p[0], p[1], biases[-1], uidx, iidx)
    return (user_out, item_out)


# speedup vs baseline: 1.1157x; 1.1157x over previous
import jax
import jax.numpy as jnp
from jax import lax
from jax.experimental import pallas as pl
from jax.experimental.pallas import tpu as pltpu
from jax.experimental.pallas import tpu_sc as plsc

N_USER = 5000
N_NODE = 10000
NPAD = 10240
D = 128
NC, NS, L = 2, 16, 16
NTILE = NC * NS
ECHUNK = 128
NCHUNK = 84
EPT = NCHUNK * ECHUNK
EPAD = NTILE * EPT
ROWS_PT = NPAD // NS


def _tc_body(x_ref, w_ref, as_ref, ad_ref, xp_ref, s_ref, d_ref):
    xp = jnp.dot(x_ref[...], w_ref[...], preferred_element_type=jnp.float32)
    xp_ref[...] = xp
    s_ref[...] = jnp.sum(xp * as_ref[...], axis=1, keepdims=True)
    d_ref[...] = jnp.sum(xp * ad_ref[...], axis=1, keepdims=True)


_TC_R = 1024


def _tc_outs():
    return dict(
        out_specs=[
            pl.BlockSpec((_TC_R, D), lambda i: (i, 0)),
            pl.BlockSpec((_TC_R, 1), lambda i: (i, 0)),
            pl.BlockSpec((_TC_R, 1), lambda i: (i, 0)),
        ],
        out_shape=[
            jax.ShapeDtypeStruct((NPAD, D), jnp.float32),
            jax.ShapeDtypeStruct((NPAD, 1), jnp.float32),
            jax.ShapeDtypeStruct((NPAD, 1), jnp.float32),
        ],
    )


def _tc_layer0(x, W, a_s, a_d):
    return pl.pallas_call(
        _tc_body,
        grid=(NPAD // _TC_R,),
        in_specs=[
            pl.BlockSpec((_TC_R, D), lambda i: (i, 0)),
            pl.BlockSpec((D, D), lambda i: (0, 0)),
            pl.BlockSpec((1, D), lambda i: (0, 0)),
            pl.BlockSpec((1, D), lambda i: (0, 0)),
        ],
        **_tc_outs(),
    )(x, W, a_s.reshape(1, D), a_d.reshape(1, D))


def _tc_body_p(p_ref, d0_ref, d1_ref, b_ref, w_ref, as_ref, ad_ref,
               xp_ref, s_ref, d_ref):
    den = d0_ref[...] + d1_ref[...] + 1e-16
    x = (p_ref[0] + p_ref[1]) / den + b_ref[...]
    xp = jnp.dot(x, w_ref[...], preferred_element_type=jnp.float32)
    xp_ref[...] = xp
    s_ref[...] = jnp.sum(xp * as_ref[...], axis=1, keepdims=True)
    d_ref[...] = jnp.sum(xp * ad_ref[...], axis=1, keepdims=True)


def _tc_layer(p, d0, d1, b, W, a_s, a_d):
    return pl.pallas_call(
        _tc_body_p,
        grid=(NPAD // _TC_R,),
        in_specs=[
            pl.BlockSpec((2, _TC_R, D), lambda i: (0, i, 0)),
            pl.BlockSpec((_TC_R, 1), lambda i: (i, 0)),
            pl.BlockSpec((_TC_R, 1), lambda i: (i, 0)),
            pl.BlockSpec((1, D), lambda i: (0, 0)),
            pl.BlockSpec((D, D), lambda i: (0, 0)),
            pl.BlockSpec((1, D), lambda i: (0, 0)),
            pl.BlockSpec((1, D), lambda i: (0, 0)),
        ],
        **_tc_outs(),
    )(p, d0, d1, b.reshape(1, D), W, a_s.reshape(1, D), a_d.reshape(1, D))


_MESH = plsc.VectorSubcoreMesh(
    core_axis_name="c", subcore_axis_name="s", num_cores=NC, num_subcores=NS)

_SC_PARAMS = pltpu.CompilerParams(needs_layout_passes=False)


def _sc_layer_body(xp_hbm, as_hbm, ad_hbm, src_hbm, dst_hbm,
                   out_hbm, den_hbm,
                   src_c, dst_c, asg_v, adg_v, exw_v, rows_v, zeros_v,
                   as_sh, ad_sh, den_sh, out_sh):
    c = lax.axis_index("c")
    s = lax.axis_index("s")
    z16 = jnp.zeros((L,), jnp.float32)

    for j in range(8):
        zeros_v[pl.ds(16 * j, 16)] = z16

    def zero_rows(r, _):
        for j in range(8):
            rows_v[0, r, pl.ds(16 * j, 16)] = z16
        return 0
    lax.fori_loop(0, ECHUNK, zero_rows, 0)

    row0 = s * ROWS_PT
    pltpu.sync_copy(as_hbm.at[pl.ds(row0, ROWS_PT)],
                    as_sh.at[pl.ds(row0, ROWS_PT)])
    pltpu.sync_copy(ad_hbm.at[pl.ds(row0, ROWS_PT)],
                    ad_sh.at[pl.ds(row0, ROWS_PT)])
    for m in range(ROWS_PT // ECHUNK):
        pltpu.sync_copy(zeros_v, den_sh.at[pl.ds(row0 + m * ECHUNK, ECHUNK)])
        pltpu.sync_copy(rows_v.at[0],
                        out_sh.at[pl.ds(row0 + m * ECHUNK, ECHUNK)])
    plsc.subcore_barrier()

    base0 = (s * NC + c) * EPT

    def scoped(**sems):
        isem = [[sems[f"i{q}{d}"] for d in range(2)] for q in range(4)]
        gsem = [[sems[f"g{k}{d}"] for d in range(3)] for k in range(2)]
        ssem = [[sems[f"s{k}{d}"] for d in range(2)] for k in range(2)]

        def issue_idx(g, q):
            base = base0 + g * ECHUNK
            pltpu.async_copy(src_hbm.at[pl.ds(base, ECHUNK)], src_c.at[q],
                             isem[q][0])
            pltpu.async_copy(dst_hbm.at[pl.ds(base, ECHUNK)], dst_c.at[q],
                             isem[q][1])

        def wait_idx(g, q):
            base = base0 + g * ECHUNK
            pltpu.make_async_copy(src_hbm.at[pl.ds(base, ECHUNK)],
                                  src_c.at[q], isem[q][0]).wait()
            pltpu.make_async_copy(dst_hbm.at[pl.ds(base, ECHUNK)],
                                  dst_c.at[q], isem[q][1]).wait()

        def issue_gathers(q, k):
            pltpu.async_copy(xp_hbm.at[src_c.at[q]], rows_v.at[k],
                             gsem[k][0])
            pltpu.async_copy(as_sh.at[src_c.at[q]], asg_v.at[k], gsem[k][1])
            pltpu.async_copy(ad_sh.at[dst_c.at[q]], adg_v.at[k], gsem[k][2])

        def wait_gathers(q, k):
            pltpu.make_async_copy(xp_hbm.at[src_c.at[q]], rows_v.at[k],
                                  gsem[k][0]).wait()
            pltpu.make_async_copy(as_sh.at[src_c.at[q]], asg_v.at[k],
                                  gsem[k][1]).wait()
            pltpu.make_async_copy(ad_sh.at[dst_c.at[q]], adg_v.at[k],
                                  gsem[k][2]).wait()

        def issue_scatters(q, k):
            pltpu.async_copy(rows_v.at[k], out_sh.at[dst_c.at[q]],
                             ssem[k][0], add=True)
            pltpu.async_copy(exw_v.at[k], den_sh.at[dst_c.at[q]],
                             ssem[k][1], add=True)

        def wait_scatters(q, k):
            pltpu.make_async_copy(rows_v.at[k], out_sh.at[dst_c.at[q]],
                                  ssem[k][0]).wait()
            pltpu.make_async_copy(exw_v.at[k], den_sh.at[dst_c.at[q]],
                                  ssem[k][1]).wait()

        def compute(k):
            for j in range(8):
                sl = pl.ds(16 * j, 16)
                al = asg_v[k, sl] + adg_v[k, sl]
                al = jnp.maximum(al, al * 0.2)
                exw_v[k, sl] = jnp.exp(al)

            def scale(e, _):
                ws = plsc.load_gather(exw_v.at[k],
                                      [jnp.full((L,), e, jnp.int32)])
                for j in range(8):
                    sl = pl.ds(16 * j, 16)
                    rows_v[k, e, sl] = rows_v[k, e, sl] * ws
                return 0
            lax.fori_loop(0, ECHUNK, scale, 0)

        issue_idx(jnp.int32(0), 0)
        issue_idx(jnp.int32(1), 1)
        wait_idx(jnp.int32(0), 0)
        issue_gathers(0, 0)

        def body4(i, _):
            for k4 in range(4):
                g = 4 * i + k4
                k = k4 % 2
                qn1 = (k4 + 1) % 4
                qn2 = (k4 + 2) % 4
                wait_gathers(k4, k)

                @pl.when(g >= 1)
                def _():
                    wait_scatters((k4 + 3) % 4, 1 - k)

                @pl.when(g + 1 < NCHUNK)
                def _():
                    wait_idx(g + 1, qn1)
                    issue_gathers(qn1, 1 - k)

                @pl.when(g + 2 < NCHUNK)
                def _():
                    issue_idx(g + 2, qn2)

                compute(k)
                issue_scatters(k4, k)
            return 0
        lax.fori_loop(0, NCHUNK // 4, body4, 0)
        wait_scatters(3, 1)

    names = ([f"i{q}{d}" for q in range(4) for d in range(2)]
             + [f"g{k}{d}" for k in range(2) for d in range(3)]
             + [f"s{k}{d}" for k in range(2) for d in range(2)])
    pl.run_scoped(scoped, **{n: pltpu.SemaphoreType.DMA(()) for n in names})
    plsc.subcore_barrier()

    pltpu.sync_copy(out_sh.at[pl.ds(row0, ROWS_PT)],
                    out_hbm.at[c].at[pl.ds(row0, ROWS_PT)])
    pltpu.sync_copy(den_sh.at[pl.ds(row0, ROWS_PT)],
                    den_hbm.at[c].at[pl.ds(row0, ROWS_PT)])


_sc_layer = pl.kernel(
    _sc_layer_body, mesh=_MESH,
    compiler_params=_SC_PARAMS,
    out_type=(jax.ShapeDtypeStruct((NC, NPAD, D), jnp.float32),
              jax.ShapeDtypeStruct((NC, NPAD), jnp.float32)),
    scratch_types=[
        pltpu.VMEM((4, ECHUNK), jnp.int32),
        pltpu.VMEM((4, ECHUNK), jnp.int32),
        pltpu.VMEM((2, ECHUNK), jnp.float32),
        pltpu.VMEM((2, ECHUNK), jnp.float32),
        pltpu.VMEM((2, ECHUNK), jnp.float32),
        pltpu.VMEM((2, ECHUNK, D), jnp.float32),
        pltpu.VMEM((ECHUNK,), jnp.float32),
        pltpu.VMEM_SHARED((NPAD,), jnp.float32),
        pltpu.VMEM_SHARED((NPAD,), jnp.float32),
        pltpu.VMEM_SHARED((NPAD,), jnp.float32),
        pltpu.VMEM_SHARED((NPAD, D), jnp.float32),
    ],
)


def _final_body(p0_hbm, p1_hbm, d0_hbm, d1_hbm, b_hbm, uidx_hbm, iidx_hbm,
                uout_hbm, iout_hbm,
                idx_v, rows0_v, rows1_v, dg0_v, dg1_v, w_v, bias_v):
    c = lax.axis_index("c")
    s = lax.axis_index("s")
    wid = s * NC + c
    pltpu.sync_copy(b_hbm, bias_v)
    b16 = [bias_v[pl.ds(16 * j, 16)] for j in range(8)]

    def one(idx_hbm, out_hbm, offset):
        pltpu.sync_copy(idx_hbm.at[wid], idx_v)
        if offset:
            for g in range(8):
                idx_v[pl.ds(16 * g, 16)] = idx_v[pl.ds(16 * g, 16)] + offset
        pltpu.sync_copy(p0_hbm.at[idx_v], rows0_v)
        pltpu.sync_copy(p1_hbm.at[idx_v], rows1_v)
        pltpu.sync_copy(d0_hbm.at[idx_v], dg0_v)
        pltpu.sync_copy(d1_hbm.at[idx_v], dg1_v)
        for j in range(8):
            sl = pl.ds(16 * j, 16)
            w_v[sl] = 1.0 / (dg0_v[sl] + dg1_v[sl] + 1e-16)

        def addrow(r, _):
            ws = plsc.load_gather(w_v, [jnp.full((L,), r, jnp.int32)])
            for j in range(8):
                sl = pl.ds(16 * j, 16)
                rows0_v[r, sl] = (rows0_v[r, sl] + rows1_v[r, sl]) * ws + b16[j]
            return 0
        lax.fori_loop(0, 128, addrow, 0)
        pltpu.sync_copy(rows0_v, out_hbm.at[pl.ds(wid * 128, 128)])

    one(uidx_hbm, uout_hbm, 0)
    one(iidx_hbm, iout_hbm, N_USER)


_final_gather = pl.kernel(
    _final_body, mesh=_MESH,
    compiler_params=_SC_PARAMS,
    out_type=(jax.ShapeDtypeStruct((4096, D), jnp.float32),
              jax.ShapeDtypeStruct((4096, D), jnp.float32)),
    scratch_types=[
        pltpu.VMEM((128,), jnp.int32),
        pltpu.VMEM((128, D), jnp.float32),
        pltpu.VMEM((128, D), jnp.float32),
        pltpu.VMEM((128,), jnp.float32),
        pltpu.VMEM((128,), jnp.float32),
        pltpu.VMEM((128,), jnp.float32),
        pltpu.VMEM((D,), jnp.float32),
    ],
)


def kernel(user_emb, item_emb, Ws, att_src, att_dst, biases, edge_index, user, item):
    x0 = jnp.concatenate([user_emb, item_emb], axis=0)
    x0 = jnp.pad(x0, ((0, NPAD - N_NODE), (0, 0)))

    loops = jnp.arange(N_NODE, dtype=jnp.int32)
    fill = jnp.full((EPAD - N_NODE - edge_index.shape[1],), N_NODE, jnp.int32)
    src_r = jnp.concatenate([edge_index[0].astype(jnp.int32), loops, fill])
    dst_r = jnp.concatenate([edge_index[1].astype(jnp.int32), loops, fill])

    xp, as2, ad2 = _tc_layer0(x0, Ws[0], att_src[0], att_dst[0])
    p, dn = _sc_layer(xp, as2.reshape(NPAD), ad2.reshape(NPAD), src_r, dst_r)
    for l in range(1, Ws.shape[0]):
        xp, as2, ad2 = _tc_layer(p, dn[0].reshape(NPAD, 1),
                                 dn[1].reshape(NPAD, 1), biases[l - 1],
                                 Ws[l], att_src[l], att_dst[l])
        p, dn = _sc_layer(xp, as2.reshape(NPAD), ad2.reshape(NPAD),
                          src_r, dst_r)

    uidx = user.astype(jnp.int32).reshape(NTILE, 128)
    iidx = item.astype(jnp.int32).reshape(NTILE, 128)
    user_out, item_out = _final_gather(p[0], p[1], dn[0], dn[1],
                                       biases[-1], uidx, iidx)
    return (user_out, item_out)


# speedup vs baseline: 1.1158x; 1.0001x over previous
import jax
import jax.numpy as jnp
from jax import lax
from jax.experimental import pallas as pl
from jax.experimental.pallas import tpu as pltpu
from jax.experimental.pallas import tpu_sc as plsc

N_USER = 5000
N_NODE = 10000
NPAD = 10240
D = 128
NC, NS, L = 2, 16, 16
NTILE = NC * NS
ECHUNK = 128
NCHUNK = 84
EPT = NCHUNK * ECHUNK
EPAD = NTILE * EPT
ROWS_PT = NPAD // NS


def _tc_body(x_ref, w_ref, as_ref, ad_ref, xp_ref, s_ref, d_ref):
    xp = jnp.dot(x_ref[...], w_ref[...], preferred_element_type=jnp.float32)
    xp_ref[...] = xp
    s_ref[...] = jnp.sum(xp * as_ref[...], axis=1, keepdims=True)
    d_ref[...] = jnp.sum(xp * ad_ref[...], axis=1, keepdims=True)


_TC_R = 1024


def _tc_outs():
    return dict(
        out_specs=[
            pl.BlockSpec((_TC_R, D), lambda i: (i, 0)),
            pl.BlockSpec((_TC_R, 1), lambda i: (i, 0)),
            pl.BlockSpec((_TC_R, 1), lambda i: (i, 0)),
        ],
        out_shape=[
            jax.ShapeDtypeStruct((NPAD, D), jnp.float32),
            jax.ShapeDtypeStruct((NPAD, 1), jnp.float32),
            jax.ShapeDtypeStruct((NPAD, 1), jnp.float32),
        ],
    )


def _tc_layer0(x, W, a_s, a_d):
    return pl.pallas_call(
        _tc_body,
        grid=(NPAD // _TC_R,),
        in_specs=[
            pl.BlockSpec((_TC_R, D), lambda i: (i, 0)),
            pl.BlockSpec((D, D), lambda i: (0, 0)),
            pl.BlockSpec((1, D), lambda i: (0, 0)),
            pl.BlockSpec((1, D), lambda i: (0, 0)),
        ],
        **_tc_outs(),
    )(x, W, a_s.reshape(1, D), a_d.reshape(1, D))


def _tc_body_p(p_ref, d0_ref, d1_ref, b_ref, w_ref, as_ref, ad_ref,
               xp_ref, s_ref, d_ref):
    den = d0_ref[...] + d1_ref[...] + 1e-16
    x = (p_ref[0] + p_ref[1]) / den + b_ref[...]
    xp = jnp.dot(x, w_ref[...], preferred_element_type=jnp.float32)
    xp_ref[...] = xp
    s_ref[...] = jnp.sum(xp * as_ref[...], axis=1, keepdims=True)
    d_ref[...] = jnp.sum(xp * ad_ref[...], axis=1, keepdims=True)


def _tc_layer(p, d0, d1, b, W, a_s, a_d):
    return pl.pallas_call(
        _tc_body_p,
        grid=(NPAD // _TC_R,),
        in_specs=[
            pl.BlockSpec((2, _TC_R, D), lambda i: (0, i, 0)),
            pl.BlockSpec((_TC_R, 1), lambda i: (i, 0)),
            pl.BlockSpec((_TC_R, 1), lambda i: (i, 0)),
            pl.BlockSpec((1, D), lambda i: (0, 0)),
            pl.BlockSpec((D, D), lambda i: (0, 0)),
            pl.BlockSpec((1, D), lambda i: (0, 0)),
            pl.BlockSpec((1, D), lambda i: (0, 0)),
        ],
        **_tc_outs(),
    )(p, d0, d1, b.reshape(1, D), W, a_s.reshape(1, D), a_d.reshape(1, D))


_MESH = plsc.VectorSubcoreMesh(
    core_axis_name="c", subcore_axis_name="s", num_cores=NC, num_subcores=NS)

_SC_PARAMS = pltpu.CompilerParams(needs_layout_passes=False)


def _sc_layer_body(xp_hbm, as_hbm, ad_hbm, src_hbm, dst_hbm,
                   out_hbm, den_hbm,
                   src_c, dst_c, asg_v, adg_v, exw_v, rows_v, zeros_v,
                   as_sh, ad_sh, den_sh, out_sh):
    c = lax.axis_index("c")
    s = lax.axis_index("s")
    z16 = jnp.zeros((L,), jnp.float32)

    for j in range(8):
        zeros_v[pl.ds(16 * j, 16)] = z16

    def zero_rows(r, _):
        for j in range(8):
            rows_v[0, r, pl.ds(16 * j, 16)] = z16
        return 0
    lax.fori_loop(0, ECHUNK, zero_rows, 0)

    row0 = s * ROWS_PT
    pltpu.sync_copy(as_hbm.at[pl.ds(row0, ROWS_PT)],
                    as_sh.at[pl.ds(row0, ROWS_PT)])
    pltpu.sync_copy(ad_hbm.at[pl.ds(row0, ROWS_PT)],
                    ad_sh.at[pl.ds(row0, ROWS_PT)])
    for m in range(ROWS_PT // ECHUNK):
        pltpu.sync_copy(zeros_v, den_sh.at[pl.ds(row0 + m * ECHUNK, ECHUNK)])
        pltpu.sync_copy(rows_v.at[0],
                        out_sh.at[pl.ds(row0 + m * ECHUNK, ECHUNK)])
    plsc.subcore_barrier()

    base0 = (s * NC + c) * EPT

    def scoped(**sems):
        isem = [[sems[f"i{q}{d}"] for d in range(2)] for q in range(4)]
        gsem = [[sems[f"g{k}{d}"] for d in range(3)] for k in range(2)]
        ssem = [[sems[f"s{k}{d}"] for d in range(2)] for k in range(2)]

        def issue_idx(g, q):
            base = base0 + g * ECHUNK
            pltpu.async_copy(src_hbm.at[pl.ds(base, ECHUNK)], src_c.at[q],
                             isem[q][0])
            pltpu.async_copy(dst_hbm.at[pl.ds(base, ECHUNK)], dst_c.at[q],
                             isem[q][1])

        def wait_idx(g, q):
            base = base0 + g * ECHUNK
            pltpu.make_async_copy(src_hbm.at[pl.ds(base, ECHUNK)],
                                  src_c.at[q], isem[q][0]).wait()
            pltpu.make_async_copy(dst_hbm.at[pl.ds(base, ECHUNK)],
                                  dst_c.at[q], isem[q][1]).wait()

        def issue_gathers(q, k):
            pltpu.async_copy(xp_hbm.at[src_c.at[q]], rows_v.at[k],
                             gsem[k][0])
            pltpu.async_copy(as_sh.at[src_c.at[q]], asg_v.at[k], gsem[k][1])
            pltpu.async_copy(ad_sh.at[dst_c.at[q]], adg_v.at[k], gsem[k][2])

        def wait_gathers(q, k):
            pltpu.make_async_copy(xp_hbm.at[src_c.at[q]], rows_v.at[k],
                                  gsem[k][0]).wait()
            pltpu.make_async_copy(as_sh.at[src_c.at[q]], asg_v.at[k],
                                  gsem[k][1]).wait()
            pltpu.make_async_copy(ad_sh.at[dst_c.at[q]], adg_v.at[k],
                                  gsem[k][2]).wait()

        def issue_scatters(q, k):
            pltpu.async_copy(rows_v.at[k], out_sh.at[dst_c.at[q]],
                             ssem[k][0], add=True)
            pltpu.async_copy(exw_v.at[k], den_sh.at[dst_c.at[q]],
                             ssem[k][1], add=True)

        def wait_scatters(q, k):
            pltpu.make_async_copy(rows_v.at[k], out_sh.at[dst_c.at[q]],
                                  ssem[k][0]).wait()
            pltpu.make_async_copy(exw_v.at[k], den_sh.at[dst_c.at[q]],
                                  ssem[k][1]).wait()

        def compute(k):
            for j in range(8):
                sl = pl.ds(16 * j, 16)
                al = asg_v[k, sl] + adg_v[k, sl]
                al = jnp.maximum(al, al * 0.2)
                exw_v[k, sl] = jnp.exp(al)

            @plsc.parallel_loop(0, ECHUNK, 1, unroll=4)
            def scale(e):
                ws = plsc.load_gather(exw_v.at[k],
                                      [jnp.full((L,), e, jnp.int32)])
                for j in range(8):
                    sl = pl.ds(16 * j, 16)
                    rows_v[k, e, sl] = rows_v[k, e, sl] * ws

        issue_idx(jnp.int32(0), 0)
        issue_idx(jnp.int32(1), 1)
        wait_idx(jnp.int32(0), 0)
        issue_gathers(0, 0)

        def body4(i, _):
            for k4 in range(4):
                g = 4 * i + k4
                k = k4 % 2
                qn1 = (k4 + 1) % 4
                qn2 = (k4 + 2) % 4
                wait_gathers(k4, k)

                @pl.when(g >= 1)
                def _():
                    wait_scatters((k4 + 3) % 4, 1 - k)

                @pl.when(g + 1 < NCHUNK)
                def _():
                    wait_idx(g + 1, qn1)
                    issue_gathers(qn1, 1 - k)

                @pl.when(g + 2 < NCHUNK)
                def _():
                    issue_idx(g + 2, qn2)

                compute(k)
                issue_scatters(k4, k)
            return 0
        lax.fori_loop(0, NCHUNK // 4, body4, 0)
        wait_scatters(3, 1)

    names = ([f"i{q}{d}" for q in range(4) for d in range(2)]
             + [f"g{k}{d}" for k in range(2) for d in range(3)]
             + [f"s{k}{d}" for k in range(2) for d in range(2)])
    pl.run_scoped(scoped, **{n: pltpu.SemaphoreType.DMA(()) for n in names})
    plsc.subcore_barrier()

    pltpu.sync_copy(out_sh.at[pl.ds(row0, ROWS_PT)],
                    out_hbm.at[c].at[pl.ds(row0, ROWS_PT)])
    pltpu.sync_copy(den_sh.at[pl.ds(row0, ROWS_PT)],
                    den_hbm.at[c].at[pl.ds(row0, ROWS_PT)])


_sc_layer = pl.kernel(
    _sc_layer_body, mesh=_MESH,
    compiler_params=_SC_PARAMS,
    out_type=(jax.ShapeDtypeStruct((NC, NPAD, D), jnp.float32),
              jax.ShapeDtypeStruct((NC, NPAD), jnp.float32)),
    scratch_types=[
        pltpu.VMEM((4, ECHUNK), jnp.int32),
        pltpu.VMEM((4, ECHUNK), jnp.int32),
        pltpu.VMEM((2, ECHUNK), jnp.float32),
        pltpu.VMEM((2, ECHUNK), jnp.float32),
        pltpu.VMEM((2, ECHUNK), jnp.float32),
        pltpu.VMEM((2, ECHUNK, D), jnp.float32),
        pltpu.VMEM((ECHUNK,), jnp.float32),
        pltpu.VMEM_SHARED((NPAD,), jnp.float32),
        pltpu.VMEM_SHARED((NPAD,), jnp.float32),
        pltpu.VMEM_SHARED((NPAD,), jnp.float32),
        pltpu.VMEM_SHARED((NPAD, D), jnp.float32),
    ],
)


def _final_body(p0_hbm, p1_hbm, d0_hbm, d1_hbm, b_hbm, uidx_hbm, iidx_hbm,
                uout_hbm, iout_hbm,
                idx_v, rows0_v, rows1_v, dg0_v, dg1_v, w_v, bias_v):
    c = lax.axis_index("c")
    s = lax.axis_index("s")
    wid = s * NC + c
    pltpu.sync_copy(b_hbm, bias_v)
    b16 = [bias_v[pl.ds(16 * j, 16)] for j in range(8)]

    def one(idx_hbm, out_hbm, offset):
        pltpu.sync_copy(idx_hbm.at[wid], idx_v)
        if offset:
            for g in range(8):
                idx_v[pl.ds(16 * g, 16)] = idx_v[pl.ds(16 * g, 16)] + offset
        pltpu.sync_copy(p0_hbm.at[idx_v], rows0_v)
        pltpu.sync_copy(p1_hbm.at[idx_v], rows1_v)
        pltpu.sync_copy(d0_hbm.at[idx_v], dg0_v)
        pltpu.sync_copy(d1_hbm.at[idx_v], dg1_v)
        for j in range(8):
            sl = pl.ds(16 * j, 16)
            w_v[sl] = 1.0 / (dg0_v[sl] + dg1_v[sl] + 1e-16)

        @plsc.parallel_loop(0, 128, 1, unroll=4)
        def addrow(r):
            ws = plsc.load_gather(w_v, [jnp.full((L,), r, jnp.int32)])
            for j in range(8):
                sl = pl.ds(16 * j, 16)
                rows0_v[r, sl] = (rows0_v[r, sl] + rows1_v[r, sl]) * ws + b16[j]
        pltpu.sync_copy(rows0_v, out_hbm.at[pl.ds(wid * 128, 128)])

    one(uidx_hbm, uout_hbm, 0)
    one(iidx_hbm, iout_hbm, N_USER)


_final_gather = pl.kernel(
    _final_body, mesh=_MESH,
    compiler_params=_SC_PARAMS,
    out_type=(jax.ShapeDtypeStruct((4096, D), jnp.float32),
              jax.ShapeDtypeStruct((4096, D), jnp.float32)),
    scratch_types=[
        pltpu.VMEM((128,), jnp.int32),
        pltpu.VMEM((128, D), jnp.float32),
        pltpu.VMEM((128, D), jnp.float32),
        pltpu.VMEM((128,), jnp.float32),
        pltpu.VMEM((128,), jnp.float32),
        pltpu.VMEM((128,), jnp.float32),
        pltpu.VMEM((D,), jnp.float32),
    ],
)


def kernel(user_emb, item_emb, Ws, att_src, att_dst, biases, edge_index, user, item):
    x0 = jnp.concatenate([user_emb, item_emb], axis=0)
    x0 = jnp.pad(x0, ((0, NPAD - N_NODE), (0, 0)))

    loops = jnp.arange(N_NODE, dtype=jnp.int32)
    fill = jnp.full((EPAD - N_NODE - edge_index.shape[1],), N_NODE, jnp.int32)
    src_r = jnp.concatenate([edge_index[0].astype(jnp.int32), loops, fill])
    dst_r = jnp.concatenate([edge_index[1].astype(jnp.int32), loops, fill])

    xp, as2, ad2 = _tc_layer0(x0, Ws[0], att_src[0], att_dst[0])
    p, dn = _sc_layer(xp, as2.reshape(NPAD), ad2.reshape(NPAD), src_r, dst_r)
    for l in range(1, Ws.shape[0]):
        xp, as2, ad2 = _tc_layer(p, dn[0].reshape(NPAD, 1),
                                 dn[1].reshape(NPAD, 1), biases[l - 1],
                                 Ws[l], att_src[l], att_dst[l])
        p, dn = _sc_layer(xp, as2.reshape(NPAD), ad2.reshape(NPAD),
                          src_r, dst_r)

    uidx = user.astype(jnp.int32).reshape(NTILE, 128)
    iidx = item.astype(jnp.int32).reshape(NTILE, 128)
    user_out, item_out = _final_gather(p[0], p[1], dn[0], dn[1],
                                       biases[-1], uidx, iidx)
    return (user_out, item_out)


# speedup vs baseline: 4.2446x; 3.8041x over previous
import jax
import jax.numpy as jnp
from jax import lax
from jax.experimental import pallas as pl
from jax.experimental.pallas import tpu as pltpu
from jax.experimental.pallas import tpu_sc as plsc

N_USER = 5000
N_NODE = 10000
NPAD = 10240
D = 128
NC, NS, L = 2, 16, 16
NTILE = NC * NS
ECHUNK = 128
NCHUNK = 84
EPT = NCHUNK * ECHUNK
EPAD = NTILE * EPT
ROWS_PT = NPAD // NS


def _tc_body(x_ref, w_ref, as_ref, ad_ref, xp_ref, s_ref, d_ref):
    xp = jnp.dot(x_ref[...], w_ref[...], preferred_element_type=jnp.float32)
    xp_ref[...] = xp
    s_ref[...] = jnp.sum(xp * as_ref[...], axis=1, keepdims=True)
    d_ref[...] = jnp.sum(xp * ad_ref[...], axis=1, keepdims=True)


_TC_R = 1024


def _tc_outs():
    return dict(
        out_specs=[
            pl.BlockSpec((_TC_R, D), lambda i: (i, 0)),
            pl.BlockSpec((_TC_R, 1), lambda i: (i, 0)),
            pl.BlockSpec((_TC_R, 1), lambda i: (i, 0)),
        ],
        out_shape=[
            jax.ShapeDtypeStruct((NPAD, D), jnp.float32),
            jax.ShapeDtypeStruct((NPAD, 1), jnp.float32),
            jax.ShapeDtypeStruct((NPAD, 1), jnp.float32),
        ],
    )


def _tc_layer0(x, W, a_s, a_d):
    return pl.pallas_call(
        _tc_body,
        grid=(NPAD // _TC_R,),
        in_specs=[
            pl.BlockSpec((_TC_R, D), lambda i: (i, 0)),
            pl.BlockSpec((D, D), lambda i: (0, 0)),
            pl.BlockSpec((1, D), lambda i: (0, 0)),
            pl.BlockSpec((1, D), lambda i: (0, 0)),
        ],
        **_tc_outs(),
    )(x, W, a_s.reshape(1, D), a_d.reshape(1, D))


def _tc_body_p(p_ref, d0_ref, d1_ref, b_ref, w_ref, as_ref, ad_ref,
               xp_ref, s_ref, d_ref):
    den = d0_ref[...] + d1_ref[...] + 1e-16
    x = (p_ref[0] + p_ref[1]) / den + b_ref[...]
    xp = jnp.dot(x, w_ref[...], preferred_element_type=jnp.float32)
    xp_ref[...] = xp
    s_ref[...] = jnp.sum(xp * as_ref[...], axis=1, keepdims=True)
    d_ref[...] = jnp.sum(xp * ad_ref[...], axis=1, keepdims=True)


def _tc_layer(p, d0, d1, b, W, a_s, a_d):
    return pl.pallas_call(
        _tc_body_p,
        grid=(NPAD // _TC_R,),
        in_specs=[
            pl.BlockSpec((2, _TC_R, D), lambda i: (0, i, 0)),
            pl.BlockSpec((_TC_R, 1), lambda i: (i, 0)),
            pl.BlockSpec((_TC_R, 1), lambda i: (i, 0)),
            pl.BlockSpec((1, D), lambda i: (0, 0)),
            pl.BlockSpec((D, D), lambda i: (0, 0)),
            pl.BlockSpec((1, D), lambda i: (0, 0)),
            pl.BlockSpec((1, D), lambda i: (0, 0)),
        ],
        **_tc_outs(),
    )(p, d0, d1, b.reshape(1, D), W, a_s.reshape(1, D), a_d.reshape(1, D))


_MESH = plsc.VectorSubcoreMesh(
    core_axis_name="c", subcore_axis_name="s", num_cores=NC, num_subcores=NS)

_SC_PARAMS = pltpu.CompilerParams(needs_layout_passes=False)


def _sc_layer_body(xp_hbm, as_hbm, ad_hbm, src_hbm, dst_hbm,
                   out_hbm, den_hbm,
                   src_c, dst_c, asg_v, adg_v, exw_v, rows_v, zeros_v,
                   as_sh, ad_sh, den_sh, out_sh):
    c = lax.axis_index("c")
    s = lax.axis_index("s")
    z16 = jnp.zeros((L,), jnp.float32)

    for j in range(8):
        zeros_v[pl.ds(16 * j, 16)] = z16

    def zero_rows(r, _):
        for j in range(8):
            rows_v[0, r, pl.ds(16 * j, 16)] = z16
        return 0
    lax.fori_loop(0, ECHUNK, zero_rows, 0)

    row0 = s * ROWS_PT
    pltpu.sync_copy(as_hbm.at[pl.ds(row0, ROWS_PT)],
                    as_sh.at[pl.ds(row0, ROWS_PT)])
    pltpu.sync_copy(ad_hbm.at[pl.ds(row0, ROWS_PT)],
                    ad_sh.at[pl.ds(row0, ROWS_PT)])
    for m in range(ROWS_PT // ECHUNK):
        pltpu.sync_copy(zeros_v, den_sh.at[pl.ds(row0 + m * ECHUNK, ECHUNK)])
        pltpu.sync_copy(rows_v.at[0],
                        out_sh.at[pl.ds(row0 + m * ECHUNK, ECHUNK)])
    plsc.subcore_barrier()

    base0 = (s * NC + c) * EPT

    def scoped(**sems):
        isem = [[sems[f"i{q}{d}"] for d in range(2)] for q in range(4)]
        gsem = [[sems[f"g{k}{d}"] for d in range(3)] for k in range(2)]
        ssem = [[sems[f"s{k}{d}"] for d in range(2)] for k in range(2)]

        def issue_idx(g, q):
            base = base0 + g * ECHUNK
            pltpu.async_copy(src_hbm.at[pl.ds(base, ECHUNK)], src_c.at[q],
                             isem[q][0])
            pltpu.async_copy(dst_hbm.at[pl.ds(base, ECHUNK)], dst_c.at[q],
                             isem[q][1])

        def wait_idx(g, q):
            base = base0 + g * ECHUNK
            pltpu.make_async_copy(src_hbm.at[pl.ds(base, ECHUNK)],
                                  src_c.at[q], isem[q][0]).wait()
            pltpu.make_async_copy(dst_hbm.at[pl.ds(base, ECHUNK)],
                                  dst_c.at[q], isem[q][1]).wait()

        def issue_gathers(q, k):
            pltpu.async_copy(xp_hbm.at[src_c.at[q]], rows_v.at[k],
                             gsem[k][0])
            pltpu.async_copy(as_sh.at[src_c.at[q]], asg_v.at[k], gsem[k][1])
            pltpu.async_copy(ad_sh.at[dst_c.at[q]], adg_v.at[k], gsem[k][2])

        def wait_gathers(q, k):
            pltpu.make_async_copy(xp_hbm.at[src_c.at[q]], rows_v.at[k],
                                  gsem[k][0]).wait()
            pltpu.make_async_copy(as_sh.at[src_c.at[q]], asg_v.at[k],
                                  gsem[k][1]).wait()
            pltpu.make_async_copy(ad_sh.at[dst_c.at[q]], adg_v.at[k],
                                  gsem[k][2]).wait()

        def issue_scatters(q, k):
            pltpu.async_copy(rows_v.at[k], out_sh.at[dst_c.at[q]],
                             ssem[k][0], add=True)
            pltpu.async_copy(exw_v.at[k], den_sh.at[dst_c.at[q]],
                             ssem[k][1], add=True)

        def wait_scatters(q, k):
            pltpu.make_async_copy(rows_v.at[k], out_sh.at[dst_c.at[q]],
                                  ssem[k][0]).wait()
            pltpu.make_async_copy(exw_v.at[k], den_sh.at[dst_c.at[q]],
                                  ssem[k][1]).wait()

        def compute(k):
            for j in range(8):
                sl = pl.ds(16 * j, 16)
                al = asg_v[k, sl] + adg_v[k, sl]
                al = jnp.maximum(al, al * 0.2)
                exw_v[k, sl] = jnp.exp(al)

            @plsc.parallel_loop(0, ECHUNK, 1, unroll=4)
            def scale(e):
                ws = plsc.load_gather(exw_v.at[k],
                                      [jnp.full((L,), e, jnp.int32)])
                for j in range(8):
                    sl = pl.ds(16 * j, 16)
                    rows_v[k, e, sl] = rows_v[k, e, sl] * ws

        issue_idx(jnp.int32(0), 0)
        issue_idx(jnp.int32(1), 1)
        wait_idx(jnp.int32(0), 0)
        issue_gathers(0, 0)

        def body4(i, _):
            for k4 in range(4):
                g = 4 * i + k4
                k = k4 % 2
                qn1 = (k4 + 1) % 4
                qn2 = (k4 + 2) % 4
                wait_gathers(k4, k)

                @pl.when(g >= 1)
                def _():
                    wait_scatters((k4 + 3) % 4, 1 - k)

                @pl.when(g + 1 < NCHUNK)
                def _():
                    wait_idx(g + 1, qn1)
                    issue_gathers(qn1, 1 - k)

                @pl.when(g + 2 < NCHUNK)
                def _():
                    issue_idx(g + 2, qn2)

                compute(k)
                issue_scatters(k4, k)
            return 0
        lax.fori_loop(0, NCHUNK // 4, body4, 0)
        wait_scatters(3, 1)

    names = ([f"i{q}{d}" for q in range(4) for d in range(2)]
             + [f"g{k}{d}" for k in range(2) for d in range(3)]
             + [f"s{k}{d}" for k in range(2) for d in range(2)])
    pl.run_scoped(scoped, **{n: pltpu.SemaphoreType.DMA(()) for n in names})
    plsc.subcore_barrier()

    pltpu.sync_copy(out_sh.at[pl.ds(row0, ROWS_PT)],
                    out_hbm.at[c].at[pl.ds(row0, ROWS_PT)])
    pltpu.sync_copy(den_sh.at[pl.ds(row0, ROWS_PT)],
                    den_hbm.at[c].at[pl.ds(row0, ROWS_PT)])


_sc_layer = pl.kernel(
    _sc_layer_body, mesh=_MESH,
    compiler_params=_SC_PARAMS,
    out_type=(jax.ShapeDtypeStruct((NC, NPAD, D), jnp.float32),
              jax.ShapeDtypeStruct((NC, NPAD), jnp.float32)),
    scratch_types=[
        pltpu.VMEM((4, ECHUNK), jnp.int32),
        pltpu.VMEM((4, ECHUNK), jnp.int32),
        pltpu.VMEM((2, ECHUNK), jnp.float32),
        pltpu.VMEM((2, ECHUNK), jnp.float32),
        pltpu.VMEM((2, ECHUNK), jnp.float32),
        pltpu.VMEM((2, ECHUNK, D), jnp.float32),
        pltpu.VMEM((ECHUNK,), jnp.float32),
        pltpu.VMEM_SHARED((NPAD,), jnp.float32),
        pltpu.VMEM_SHARED((NPAD,), jnp.float32),
        pltpu.VMEM_SHARED((NPAD,), jnp.float32),
        pltpu.VMEM_SHARED((NPAD, D), jnp.float32),
    ],
)


def _final_body(p0_hbm, p1_hbm, d0_hbm, d1_hbm, b_hbm, uidx_hbm, iidx_hbm,
                uout_hbm, iout_hbm,
                idx_v, rows0_v, rows1_v, dg0_v, dg1_v, w_v, bias_v):
    c = lax.axis_index("c")
    s = lax.axis_index("s")
    wid = s * NC + c
    pltpu.sync_copy(b_hbm, bias_v)
    b16 = [bias_v[pl.ds(16 * j, 16)] for j in range(8)]

    def one(idx_hbm, out_hbm, offset):
        pltpu.sync_copy(idx_hbm.at[wid], idx_v)
        if offset:
            for g in range(8):
                idx_v[pl.ds(16 * g, 16)] = idx_v[pl.ds(16 * g, 16)] + offset
        pltpu.sync_copy(p0_hbm.at[idx_v], rows0_v)
        pltpu.sync_copy(p1_hbm.at[idx_v], rows1_v)
        pltpu.sync_copy(d0_hbm.at[idx_v], dg0_v)
        pltpu.sync_copy(d1_hbm.at[idx_v], dg1_v)
        for j in range(8):
            sl = pl.ds(16 * j, 16)
            w_v[sl] = 1.0 / (dg0_v[sl] + dg1_v[sl] + 1e-16)

        @plsc.parallel_loop(0, 128, 1, unroll=4)
        def addrow(r):
            ws = plsc.load_gather(w_v, [jnp.full((L,), r, jnp.int32)])
            for j in range(8):
                sl = pl.ds(16 * j, 16)
                rows0_v[r, sl] = (rows0_v[r, sl] + rows1_v[r, sl]) * ws + b16[j]
        pltpu.sync_copy(rows0_v, out_hbm.at[pl.ds(wid * 128, 128)])

    one(uidx_hbm, uout_hbm, 0)
    one(iidx_hbm, iout_hbm, N_USER)


_final_gather = pl.kernel(
    _final_body, mesh=_MESH,
    compiler_params=_SC_PARAMS,
    out_type=(jax.ShapeDtypeStruct((4096, D), jnp.float32),
              jax.ShapeDtypeStruct((4096, D), jnp.float32)),
    scratch_types=[
        pltpu.VMEM((128,), jnp.int32),
        pltpu.VMEM((128, D), jnp.float32),
        pltpu.VMEM((128, D), jnp.float32),
        pltpu.VMEM((128,), jnp.float32),
        pltpu.VMEM((128,), jnp.float32),
        pltpu.VMEM((128,), jnp.float32),
        pltpu.VMEM((D,), jnp.float32),
    ],
)


def kernel(user_emb, item_emb, Ws, att_src, att_dst, biases, edge_index, user, item):
    x0 = jnp.concatenate([user_emb, item_emb], axis=0)
    x0 = jnp.pad(x0, ((0, NPAD - N_NODE), (0, 0)))

    loops = jnp.arange(N_NODE, dtype=jnp.int32)
    n_fill = EPAD - N_NODE - edge_index.shape[1]
    fill = N_NODE + (jnp.arange(n_fill, dtype=jnp.int32) % (NPAD - N_NODE))
    src_r = jnp.concatenate([edge_index[0].astype(jnp.int32), loops, fill])
    dst_r = jnp.concatenate([edge_index[1].astype(jnp.int32), loops, fill])

    xp, as2, ad2 = _tc_layer0(x0, Ws[0], att_src[0], att_dst[0])
    p, dn = _sc_layer(xp, as2.reshape(NPAD), ad2.reshape(NPAD), src_r, dst_r)
    for l in range(1, Ws.shape[0]):
        xp, as2, ad2 = _tc_layer(p, dn[0].reshape(NPAD, 1),
                                 dn[1].reshape(NPAD, 1), biases[l - 1],
                                 Ws[l], att_src[l], att_dst[l])
        p, dn = _sc_layer(xp, as2.reshape(NPAD), ad2.reshape(NPAD),
                          src_r, dst_r)

    uidx = user.astype(jnp.int32).reshape(NTILE, 128)
    iidx = item.astype(jnp.int32).reshape(NTILE, 128)
    user_out, item_out = _final_gather(p[0], p[1], dn[0], dn[1],
                                       biases[-1], uidx, iidx)
    return (user_out, item_out)
